# baseline (speedup 1.0000x reference)
import functools

import jax
import jax.numpy as jnp
from jax import lax
from jax.experimental import pallas as pl
from jax.experimental.pallas import tpu as pltpu
from jax.experimental.pallas import tpu_sc as plsc

N = 10000
E = 320000
D_IN = 128
D = 16

NTILES = 32
EPT = 10240
EPAD = NTILES * EPT
G = EPT // 128
RING = 16
LOOK = 8
SPT = 625
NPAD = N + 8

_f32 = jnp.float32
_sc_params = pltpu.CompilerParams(use_tc_tiling_on_sc=False,
                                  needs_layout_passes=False)


def _acc_init(s, zeros_hbm, acc_sh):
    pltpu.sync_copy(zeros_hbm.at[pl.ds(0, SPT)],
                    acc_sh.at[pl.ds(s * SPT, SPT)])

    @pl.when(s == 15)
    def _():
        pltpu.sync_copy(zeros_hbm.at[pl.ds(SPT, 8)], acc_sh.at[pl.ds(N, 8)])


def _rsqrt16(x):
    i = plsc.bitcast(x, jnp.int32)
    y = plsc.bitcast(jnp.int32(0x5F3759DF) - (i >> 1), _f32)
    for _ in range(3):
        y = y * (1.5 - 0.5 * x * y * y)
    return y


def _edge_pipeline(zs_sh, acc_sh, src_v, dst_v, rows_v, gsem, ssem):
    gh = [None] * G
    sh = [None] * G
    for k in range(G + LOOK):
        if k < G:
            b = k % RING
            if k >= RING:
                sh[k - RING].wait()
            gh[k] = pltpu.async_copy(
                zs_sh.at[src_v.at[k]],
                rows_v.at[pl.ds(b * 128, 128)], gsem)
        if k >= LOOK:
            j = k - LOOK
            gh[j].wait()
            sh[j] = pltpu.async_copy(
                rows_v.at[pl.ds((j % RING) * 128, 128)],
                acc_sh.at[dst_v.at[j]], ssem, add=True)
    for j in range(G - RING, G):
        sh[j].wait()


def _flush(ref_sh, out_hbm, s, rows=SPT):
    pltpu.sync_copy(ref_sh.at[pl.ds(s * rows, rows)],
                    out_hbm.at[pl.ds(s * rows, rows)])


def _sc_degree_body(ones_hbm, dst_hbm, zeros_hbm, out_hbm,
                    dst_v, ones_v, acc_sh, ssem):
    c = lax.axis_index("c")
    s = lax.axis_index("s")
    wid = c * 16 + s

    _acc_init(s, zeros_hbm, acc_sh)
    pltpu.sync_copy(ones_hbm, ones_v)
    pltpu.sync_copy(dst_hbm.at[pl.ds(wid * G, G)], dst_v)
    plsc.subcore_barrier()

    sh = [pltpu.async_copy(ones_v, acc_sh.at[dst_v.at[k]], ssem, add=True)
          for k in range(G)]
    for h in sh:
        h.wait()

    plsc.subcore_barrier()
    _flush(acc_sh, out_hbm.at[c], s)


def _sc_agg1_body(z1_hbm, hist_hbm, src_hbm, dst_hbm, zeros_hbm,
                  p_hbm, zs1_hbm, dis_hbm,
                  src_v, dst_v, rows_v, h0_v, h1_v, z1_v, zs_v, dis_v,
                  zs_sh, acc_sh, gsem, ssem):
    c = lax.axis_index("c")
    s = lax.axis_index("s")
    wid = c * 16 + s

    _acc_init(s, zeros_hbm, acc_sh)

    base = s * SPT
    pltpu.sync_copy(hist_hbm.at[0, pl.ds(base, SPT)], h0_v)
    pltpu.sync_copy(hist_hbm.at[1, pl.ds(base, SPT)], h1_v)
    pltpu.sync_copy(z1_hbm.at[pl.ds(base, SPT)], z1_v)

    def row(i, carry):
        deg = h0_v[i] + h1_v[i] + 1.0
        d = _rsqrt16(deg)
        dis_v[i] = d
        zs_v[i] = z1_v[i] * d
        return carry

    lax.fori_loop(0, SPT, row, 0)

    pltpu.sync_copy(zs_v, zs_sh.at[pl.ds(base, SPT)])

    @pl.when(c == 0)
    def _():
        pltpu.sync_copy(zs_v, zs1_hbm.at[pl.ds(base, SPT)])
        pltpu.sync_copy(dis_v, dis_hbm.at[pl.ds(base, SPT)])

    pltpu.sync_copy(src_hbm.at[pl.ds(wid * G, G)], src_v)
    pltpu.sync_copy(dst_hbm.at[pl.ds(wid * G, G)], dst_v)
    plsc.subcore_barrier()

    _edge_pipeline(zs_sh, acc_sh, src_v, dst_v, rows_v, gsem, ssem)

    plsc.subcore_barrier()
    _flush(acc_sh, p_hbm.at[c], s)


def _sc_agg2_body(p_hbm, zs1_hbm, dis_hbm, b1_hbm, src_hbm, dst_hbm,
                  zeros_hbm, q_hbm, h1s_hbm,
                  src_v, dst_v, rows_v, p0_v, p1_v, zs1_v, dis_v, h_v, b1_v,
                  zs_sh, acc_sh, gsem, ssem):
    c = lax.axis_index("c")
    s = lax.axis_index("s")
    wid = c * 16 + s

    _acc_init(s, zeros_hbm, acc_sh)

    base = s * SPT
    pltpu.sync_copy(p_hbm.at[0, pl.ds(base, SPT)], p0_v)
    pltpu.sync_copy(p_hbm.at[1, pl.ds(base, SPT)], p1_v)
    pltpu.sync_copy(zs1_hbm.at[pl.ds(base, SPT)], zs1_v)
    pltpu.sync_copy(dis_hbm.at[pl.ds(base, SPT)], dis_v)
    pltpu.sync_copy(b1_hbm, b1_v)

    def row(i, carry):
        d = dis_v[i]
        h = d * (p0_v[i] + p1_v[i] + zs1_v[i]) + b1_v[...]
        h_v[i] = d * jnp.maximum(h, 0.0)
        return carry

    lax.fori_loop(0, SPT, row, 0)

    pltpu.sync_copy(h_v, zs_sh.at[pl.ds(base, SPT)])

    @pl.when(c == 0)
    def _():
        pltpu.sync_copy(h_v, h1s_hbm.at[pl.ds(base, SPT)])

    pltpu.sync_copy(src_hbm.at[pl.ds(wid * G, G)], src_v)
    pltpu.sync_copy(dst_hbm.at[pl.ds(wid * G, G)], dst_v)
    plsc.subcore_barrier()

    _edge_pipeline(zs_sh, acc_sh, src_v, dst_v, rows_v, gsem, ssem)

    plsc.subcore_barrier()
    _flush(acc_sh, q_hbm.at[c], s)


@functools.lru_cache(maxsize=1)
def _make_sc_kernels():
    mesh = plsc.VectorSubcoreMesh(core_axis_name="c", subcore_axis_name="s")
    deg = pl.kernel(
        _sc_degree_body,
        mesh=mesh,
        out_type=jax.ShapeDtypeStruct((2, N, D), _f32),
        scratch_types=[
            pltpu.VMEM((G, 128), jnp.int32),
            pltpu.VMEM((128, D), _f32),
            pltpu.VMEM_SHARED((NPAD, D), _f32),
            pltpu.SemaphoreType.DMA,
        ],
        compiler_params=_sc_params,
    )
    agg1 = pl.kernel(
        _sc_agg1_body,
        mesh=mesh,
        out_type=(
            jax.ShapeDtypeStruct((2, N, D), _f32),
            jax.ShapeDtypeStruct((N, D), _f32),
            jax.ShapeDtypeStruct((N, D), _f32),
        ),
        scratch_types=[
            pltpu.VMEM((G, 128), jnp.int32),
            pltpu.VMEM((G, 128), jnp.int32),
            pltpu.VMEM((RING * 128, D), _f32),
            pltpu.VMEM((SPT, D), _f32),
            pltpu.VMEM((SPT, D), _f32),
            pltpu.VMEM((SPT, D), _f32),
            pltpu.VMEM((SPT, D), _f32),
            pltpu.VMEM((SPT, D), _f32),
            pltpu.VMEM_SHARED((N, D), _f32),
            pltpu.VMEM_SHARED((NPAD, D), _f32),
            pltpu.SemaphoreType.DMA,
            pltpu.SemaphoreType.DMA,
        ],
        compiler_params=_sc_params,
    )
    agg2 = pl.kernel(
        _sc_agg2_body,
        mesh=mesh,
        out_type=(
            jax.ShapeDtypeStruct((2, N, D), _f32),
            jax.ShapeDtypeStruct((N, D), _f32),
        ),
        scratch_types=[
            pltpu.VMEM((G, 128), jnp.int32),
            pltpu.VMEM((G, 128), jnp.int32),
            pltpu.VMEM((RING * 128, D), _f32),
            pltpu.VMEM((SPT, D), _f32),
            pltpu.VMEM((SPT, D), _f32),
            pltpu.VMEM((SPT, D), _f32),
            pltpu.VMEM((SPT, D), _f32),
            pltpu.VMEM((SPT, D), _f32),
            pltpu.VMEM((D,), _f32),
            pltpu.VMEM_SHARED((N, D), _f32),
            pltpu.VMEM_SHARED((NPAD, D), _f32),
            pltpu.SemaphoreType.DMA,
            pltpu.SemaphoreType.DMA,
        ],
        compiler_params=_sc_params,
    )
    return deg, agg1, agg2


def _tc1a_body(x, w1, es, ed, z, src_pad, dst_pad, ones_t, zeros_t):
    z[...] = jnp.dot(x[...], w1[...], preferred_element_type=_f32)
    src_pad[0:E // 128, :] = es[...]
    src_pad[E // 128:, :] = jnp.zeros((60, 128), jnp.int32)
    dst_pad[0:E // 128, :] = ed[...]
    dst_pad[E // 128:, :] = jnp.full((60, 128), N, jnp.int32)
    ones_t[...] = jnp.ones((128, D), _f32)
    zeros_t[...] = jnp.zeros((SPT + 8, D), _f32)


def _tcf_body(q0, q1, h1s, dis, w2, b2, out):
    agg = q0[...] + q1[...] + h1s[...]
    out[...] = jnp.dot(agg, w2[...], preferred_element_type=_f32,
                       precision=jax.lax.Precision.HIGHEST) \
        * dis[...] + b2[...]


_row_spec = pl.BlockSpec((N, D), lambda: (0, 0))

_tc1a = pl.pallas_call(
    _tc1a_body,
    in_specs=[
        pl.BlockSpec((N, D_IN), lambda: (0, 0)),
        pl.BlockSpec((D_IN, D), lambda: (0, 0)),
        pl.BlockSpec((E // 128, 128), lambda: (0, 0)),
        pl.BlockSpec((E // 128, 128), lambda: (0, 0)),
    ],
    out_specs=[
        _row_spec,
        pl.BlockSpec((EPAD // 128, 128), lambda: (0, 0)),
        pl.BlockSpec((EPAD // 128, 128), lambda: (0, 0)),
        pl.BlockSpec((128, D), lambda: (0, 0)),
        pl.BlockSpec((SPT + 8, D), lambda: (0, 0)),
    ],
    out_shape=[
        jax.ShapeDtypeStruct((N, D), _f32),
        jax.ShapeDtypeStruct((EPAD // 128, 128), jnp.int32),
        jax.ShapeDtypeStruct((EPAD // 128, 128), jnp.int32),
        jax.ShapeDtypeStruct((128, D), _f32),
        jax.ShapeDtypeStruct((SPT + 8, D), _f32),
    ],
)

_tcf = pl.pallas_call(
    _tcf_body,
    in_specs=[
        _row_spec, _row_spec, _row_spec, _row_spec,
        pl.BlockSpec((D, D), lambda: (0, 0)),
        pl.BlockSpec((1, D), lambda: (0, 0)),
    ],
    out_specs=_row_spec,
    out_shape=jax.ShapeDtypeStruct((N, D), _f32),
)


def kernel(x, edge_index, W1, b1, W2, b2):
    es = edge_index[0].astype(jnp.int32).reshape(E // 128, 128)
    ed = edge_index[1].astype(jnp.int32).reshape(E // 128, 128)

    sc_degree, sc_agg1, sc_agg2 = _make_sc_kernels()
    z1, src_pad, dst_pad, ones_t, zeros_t = _tc1a(x, W1, es, ed)
    hist = sc_degree(ones_t, dst_pad, zeros_t)
    p, zs1, dis = sc_agg1(z1, hist, src_pad, dst_pad, zeros_t)
    q, h1s = sc_agg2(p, zs1, dis, b1, src_pad, dst_pad, zeros_t)
    return _tcf(q[0], q[1], h1s, dis, W2, b2.reshape(1, D))


# speedup vs baseline: 1.0506x; 1.0506x over previous
import functools

import jax
import jax.numpy as jnp
from jax import lax
from jax.experimental import pallas as pl
from jax.experimental.pallas import tpu as pltpu
from jax.experimental.pallas import tpu_sc as plsc

N = 10000
E = 320000
D_IN = 128
D = 16

NTILES = 32
EPT = 10240
EPAD = NTILES * EPT
G = EPT // 128
RING = 16
LOOK = 8
SPT = 625
NPAD = N + 8

_f32 = jnp.float32
_sc_params = pltpu.CompilerParams(use_tc_tiling_on_sc=False,
                                  needs_layout_passes=False)


def _acc_init(s, zeros_hbm, acc_sh):
    pltpu.sync_copy(zeros_hbm.at[pl.ds(0, SPT)],
                    acc_sh.at[pl.ds(s * SPT, SPT)])

    @pl.when(s == 15)
    def _():
        pltpu.sync_copy(zeros_hbm.at[pl.ds(SPT, 8)], acc_sh.at[pl.ds(N, 8)])


def _rsqrt16(x):
    i = plsc.bitcast(x, jnp.int32)
    y = plsc.bitcast(jnp.int32(0x5F3759DF) - (i >> 1), _f32)
    for _ in range(3):
        y = y * (1.5 - 0.5 * x * y * y)
    return y


def _edge_pipeline(zs_sh, acc_sh, src_v, dst_v, rows_v, gsem, ssem):
    gh = [None] * G
    sh = [None] * G
    for k in range(G + LOOK):
        if k < G:
            b = k % RING
            if k >= RING:
                sh[k - RING].wait()
            gh[k] = pltpu.async_copy(
                zs_sh.at[src_v.at[k]],
                rows_v.at[pl.ds(b * 128, 128)], gsem)
        if k >= LOOK:
            j = k - LOOK
            gh[j].wait()
            sh[j] = pltpu.async_copy(
                rows_v.at[pl.ds((j % RING) * 128, 128)],
                acc_sh.at[dst_v.at[j]], ssem, add=True)
    for j in range(G - RING, G):
        sh[j].wait()


def _flush(ref_sh, out_hbm, s, rows=SPT):
    pltpu.sync_copy(ref_sh.at[pl.ds(s * rows, rows)],
                    out_hbm.at[pl.ds(s * rows, rows)])


def _sc_degree_body(ones_hbm, dst_hbm, zeros_hbm, out_hbm,
                    dst_v, ones_v, acc_sh, ssem):
    c = lax.axis_index("c")
    s = lax.axis_index("s")
    wid = c * 16 + s

    _acc_init(s, zeros_hbm, acc_sh)
    pltpu.sync_copy(ones_hbm, ones_v)
    pltpu.sync_copy(dst_hbm.at[pl.ds(wid * G, G)], dst_v)
    plsc.subcore_barrier()

    sh = [pltpu.async_copy(ones_v, acc_sh.at[dst_v.at[k]], ssem, add=True)
          for k in range(G)]
    for h in sh:
        h.wait()

    plsc.subcore_barrier()
    _flush(acc_sh, out_hbm.at[c], s)


def _sc_agg1_body(z1_hbm, hist_hbm, src_hbm, dst_hbm, zeros_hbm,
                  p_hbm, zs1_hbm, dis_hbm,
                  src_v, dst_v, rows_v, h0_v, h1_v, z1_v, zs_v, dis_v,
                  zs_sh, acc_sh, gsem, ssem):
    c = lax.axis_index("c")
    s = lax.axis_index("s")
    wid = c * 16 + s

    _acc_init(s, zeros_hbm, acc_sh)

    base = s * SPT
    pltpu.sync_copy(hist_hbm.at[0, pl.ds(base, SPT)], h0_v)
    pltpu.sync_copy(hist_hbm.at[1, pl.ds(base, SPT)], h1_v)
    pltpu.sync_copy(z1_hbm.at[pl.ds(base, SPT)], z1_v)

    def row(i, carry):
        deg = h0_v[i] + h1_v[i] + 1.0
        d = _rsqrt16(deg)
        dis_v[i] = d
        zs_v[i] = z1_v[i] * d
        return carry

    lax.fori_loop(0, SPT, row, 0)

    pltpu.sync_copy(zs_v, zs_sh.at[pl.ds(base, SPT)])

    @pl.when(c == 0)
    def _():
        pltpu.sync_copy(zs_v, zs1_hbm.at[pl.ds(base, SPT)])
        pltpu.sync_copy(dis_v, dis_hbm.at[pl.ds(base, SPT)])

    pltpu.sync_copy(src_hbm.at[pl.ds(wid * G, G)], src_v)
    pltpu.sync_copy(dst_hbm.at[pl.ds(wid * G, G)], dst_v)
    plsc.subcore_barrier()

    _edge_pipeline(zs_sh, acc_sh, src_v, dst_v, rows_v, gsem, ssem)

    plsc.subcore_barrier()
    _flush(acc_sh, p_hbm.at[c], s)


def _sc_agg2_body(p_hbm, zs1_hbm, dis_hbm, b1_hbm, src_hbm, dst_hbm,
                  zeros_hbm, q_hbm, h1s_hbm,
                  src_v, dst_v, rows_v, p0_v, p1_v, zs1_v, dis_v, h_v, b1_v,
                  zs_sh, acc_sh, gsem, ssem):
    c = lax.axis_index("c")
    s = lax.axis_index("s")
    wid = c * 16 + s

    _acc_init(s, zeros_hbm, acc_sh)

    base = s * SPT
    pltpu.sync_copy(p_hbm.at[0, pl.ds(base, SPT)], p0_v)
    pltpu.sync_copy(p_hbm.at[1, pl.ds(base, SPT)], p1_v)
    pltpu.sync_copy(zs1_hbm.at[pl.ds(base, SPT)], zs1_v)
    pltpu.sync_copy(dis_hbm.at[pl.ds(base, SPT)], dis_v)
    pltpu.sync_copy(b1_hbm, b1_v)

    def row(i, carry):
        d = dis_v[i]
        h = d * (p0_v[i] + p1_v[i] + zs1_v[i]) + b1_v[...]
        h_v[i] = d * jnp.maximum(h, 0.0)
        return carry

    lax.fori_loop(0, SPT, row, 0)

    pltpu.sync_copy(h_v, zs_sh.at[pl.ds(base, SPT)])

    @pl.when(c == 0)
    def _():
        pltpu.sync_copy(h_v, h1s_hbm.at[pl.ds(base, SPT)])

    pltpu.sync_copy(src_hbm.at[pl.ds(wid * G, G)], src_v)
    pltpu.sync_copy(dst_hbm.at[pl.ds(wid * G, G)], dst_v)
    plsc.subcore_barrier()

    _edge_pipeline(zs_sh, acc_sh, src_v, dst_v, rows_v, gsem, ssem)

    plsc.subcore_barrier()
    _flush(acc_sh, q_hbm.at[c], s)


@functools.lru_cache(maxsize=1)
def _make_sc_kernels():
    mesh = plsc.VectorSubcoreMesh(core_axis_name="c", subcore_axis_name="s")
    deg = pl.kernel(
        _sc_degree_body,
        mesh=mesh,
        out_type=jax.ShapeDtypeStruct((2, N, D), _f32),
        scratch_types=[
            pltpu.VMEM((G, 128), jnp.int32),
            pltpu.VMEM((128, D), _f32),
            pltpu.VMEM_SHARED((NPAD, D), _f32),
            pltpu.SemaphoreType.DMA,
        ],
        compiler_params=_sc_params,
    )
    agg1 = pl.kernel(
        _sc_agg1_body,
        mesh=mesh,
        out_type=(
            jax.ShapeDtypeStruct((2, N, D), _f32),
            jax.ShapeDtypeStruct((N, D), _f32),
            jax.ShapeDtypeStruct((N, D), _f32),
        ),
        scratch_types=[
            pltpu.VMEM((G, 128), jnp.int32),
            pltpu.VMEM((G, 128), jnp.int32),
            pltpu.VMEM((RING * 128, D), _f32),
            pltpu.VMEM((SPT, D), _f32),
            pltpu.VMEM((SPT, D), _f32),
            pltpu.VMEM((SPT, D), _f32),
            pltpu.VMEM((SPT, D), _f32),
            pltpu.VMEM((SPT, D), _f32),
            pltpu.VMEM_SHARED((N, D), _f32),
            pltpu.VMEM_SHARED((NPAD, D), _f32),
            pltpu.SemaphoreType.DMA,
            pltpu.SemaphoreType.DMA,
        ],
        compiler_params=_sc_params,
    )
    agg2 = pl.kernel(
        _sc_agg2_body,
        mesh=mesh,
        out_type=(
            jax.ShapeDtypeStruct((2, N, D), _f32),
            jax.ShapeDtypeStruct((N, D), _f32),
        ),
        scratch_types=[
            pltpu.VMEM((G, 128), jnp.int32),
            pltpu.VMEM((G, 128), jnp.int32),
            pltpu.VMEM((RING * 128, D), _f32),
            pltpu.VMEM((SPT, D), _f32),
            pltpu.VMEM((SPT, D), _f32),
            pltpu.VMEM((SPT, D), _f32),
            pltpu.VMEM((SPT, D), _f32),
            pltpu.VMEM((SPT, D), _f32),
            pltpu.VMEM((D,), _f32),
            pltpu.VMEM_SHARED((N, D), _f32),
            pltpu.VMEM_SHARED((NPAD, D), _f32),
            pltpu.SemaphoreType.DMA,
            pltpu.SemaphoreType.DMA,
        ],
        compiler_params=_sc_params,
    )
    return deg, agg1, agg2


def _tc1a_body(x, w1, z):
    z[...] = jnp.dot(x[...], w1[...], preferred_element_type=_f32)


def _tcf_body(q0, q1, h1s, dis, w2, b2, out):
    agg = q0[...] + q1[...] + h1s[...]
    out[...] = jnp.dot(agg, w2[...], preferred_element_type=_f32,
                       precision=jax.lax.Precision.HIGHEST) \
        * dis[...] + b2[...]


_row_spec = pl.BlockSpec((N, D), lambda: (0, 0))

_tc1a = pl.pallas_call(
    _tc1a_body,
    in_specs=[
        pl.BlockSpec((N, D_IN), lambda: (0, 0)),
        pl.BlockSpec((D_IN, D), lambda: (0, 0)),
    ],
    out_specs=_row_spec,
    out_shape=jax.ShapeDtypeStruct((N, D), _f32),
)

_tcf = pl.pallas_call(
    _tcf_body,
    in_specs=[
        _row_spec, _row_spec, _row_spec, _row_spec,
        pl.BlockSpec((D, D), lambda: (0, 0)),
        pl.BlockSpec((1, D), lambda: (0, 0)),
    ],
    out_specs=_row_spec,
    out_shape=jax.ShapeDtypeStruct((N, D), _f32),
)


def kernel(x, edge_index, W1, b1, W2, b2):
    src = edge_index[0].astype(jnp.int32)
    dst = edge_index[1].astype(jnp.int32)
    npad = EPAD - E
    src_pad = jnp.concatenate(
        [src, jnp.zeros((npad,), jnp.int32)]).reshape(EPAD // 128, 128)
    dst_pad = jnp.concatenate(
        [dst, jnp.full((npad,), N, jnp.int32)]).reshape(EPAD // 128, 128)
    ones_t = jnp.ones((128, D), _f32)
    zeros_t = jnp.zeros((SPT + 8, D), _f32)

    sc_degree, sc_agg1, sc_agg2 = _make_sc_kernels()
    z1 = _tc1a(x, W1)
    hist = sc_degree(ones_t, dst_pad, zeros_t)
    p, zs1, dis = sc_agg1(z1, hist, src_pad, dst_pad, zeros_t)
    q, h1s = sc_agg2(p, zs1, dis, b1, src_pad, dst_pad, zeros_t)
    return _tcf(q[0], q[1], h1s, dis, W2, b2.reshape(1, D))
